# baseline (speedup 1.0000x reference)
import jax
import jax.numpy as jnp
from jax import lax
from jax.experimental import pallas as pl
from jax.experimental.pallas import tpu as pltpu
from jax.experimental.pallas import tpu_sc as plsc

NC = 2
NS = 16
NW = NC * NS
LANES = 16
GSZ = 128
SGSZ = 128
NBUF = 2
SPLIT = 4


def _feat_body(x_ref, w_ref, b_ref, o_ref):
    acc = jnp.dot(x_ref[...], w_ref[...], preferred_element_type=jnp.float32)
    o_ref[...] = jnp.maximum(acc + b_ref[...], 0.0)


def _tc_feat(x, w, b):
    n, d = x.shape
    h_dim = w.shape[1]
    bn = 1000
    return pl.pallas_call(
        _feat_body,
        grid=(n // bn,),
        in_specs=[
            pl.BlockSpec((bn, d), lambda i: (i, 0)),
            pl.BlockSpec((d, h_dim), lambda i: (0, 0)),
            pl.BlockSpec((1, h_dim), lambda i: (0, 0)),
        ],
        out_specs=pl.BlockSpec((bn, h_dim), lambda i: (i, 0)),
        out_shape=jax.ShapeDtypeStruct((n, h_dim), jnp.float32),
    )(x, w, b.reshape(1, -1))


def _mid_body(s_ref, c_ref, w_ref, b_ref, o_ref):
    cnt = c_ref[0, :, :1] + c_ref[1, :, :1]
    m = (s_ref[0] + s_ref[1]) / jnp.maximum(cnt, 1.0)
    acc = jnp.dot(m, w_ref[...], preferred_element_type=jnp.float32)
    o_ref[...] = jnp.maximum(acc + b_ref[...], 0.0)


def _tc_mid(sums, cnts, w, b):
    _, n_pad, h_dim = sums.shape
    bn = n_pad // 4
    return pl.pallas_call(
        _mid_body,
        grid=(4,),
        in_specs=[
            pl.BlockSpec((2, bn, h_dim), lambda i: (0, i, 0)),
            pl.BlockSpec((2, bn, h_dim), lambda i: (0, i, 0)),
            pl.BlockSpec((h_dim, h_dim), lambda i: (0, 0)),
            pl.BlockSpec((1, h_dim), lambda i: (0, 0)),
        ],
        out_specs=pl.BlockSpec((bn, h_dim), lambda i: (i, 0)),
        out_shape=jax.ShapeDtypeStruct((n_pad, h_dim), jnp.float32),
    )(sums, cnts, w, b.reshape(1, -1))


def _final_body(s_ref, c_ref, w2_ref, b2_ref, ge_ref, wc1_ref, bc1_ref,
                wc2_ref, bc2_ref, o_ref):
    cnt = c_ref[0, :, :1] + c_ref[1, :, :1]
    m = (s_ref[0] + s_ref[1]) / jnp.maximum(cnt, 1.0)
    h2 = jnp.dot(m, w2_ref[...], preferred_element_type=jnp.float32) + b2_ref[...]
    h2 = h2 + jnp.mean(ge_ref[...], axis=0, keepdims=True)
    hid = jnp.dot(h2, wc1_ref[...], preferred_element_type=jnp.float32) + bc1_ref[...]
    hid = jnp.maximum(hid, 0.0)
    o_ref[...] = jnp.dot(hid, wc2_ref[...], preferred_element_type=jnp.float32) + bc2_ref[...]


def _tc_final(sums, cnts, w2, b2, ge, wc1, bc1, wc2, bc2):
    _, n_pad, h_dim = sums.shape
    g = ge.shape[0]
    hid_dim = wc1.shape[1]
    out_dim = wc2.shape[1]
    bn = n_pad // 4
    return pl.pallas_call(
        _final_body,
        grid=(4,),
        in_specs=[
            pl.BlockSpec((2, bn, h_dim), lambda i: (0, i, 0)),
            pl.BlockSpec((2, bn, h_dim), lambda i: (0, i, 0)),
            pl.BlockSpec((h_dim, h_dim), lambda i: (0, 0)),
            pl.BlockSpec((1, h_dim), lambda i: (0, 0)),
            pl.BlockSpec((g, h_dim), lambda i: (0, 0)),
            pl.BlockSpec((h_dim, hid_dim), lambda i: (0, 0)),
            pl.BlockSpec((1, hid_dim), lambda i: (0, 0)),
            pl.BlockSpec((hid_dim, out_dim), lambda i: (0, 0)),
            pl.BlockSpec((1, out_dim), lambda i: (0, 0)),
        ],
        out_specs=pl.BlockSpec((bn, out_dim), lambda i: (i, 0)),
        out_shape=jax.ShapeDtypeStruct((n_pad, out_dim), jnp.float32),
    )(sums, cnts, w2, b2.reshape(1, -1), ge, wc1, bc1.reshape(1, -1),
      wc2, bc2.reshape(1, -1))


def _mesh():
    return plsc.VectorSubcoreMesh(
        core_axis_name="c", subcore_axis_name="s",
        num_cores=NC, num_subcores=NS)


def _sc_counts(dst4, n_pad, h_dim):
    k4 = dst4.shape[2]
    rpt = n_pad // NS
    n_full = rpt // GSZ
    rem = rpt - n_full * GSZ

    def body(dst_hbm, cnts_hbm, dst_v, ones_v, cnt_acc, sem):
        del sem
        core = lax.axis_index("c")
        sub = lax.axis_index("s")
        wid = sub * NC + core
        base = sub * rpt

        zero16 = jnp.zeros((LANES,), jnp.float32)
        one16 = jnp.ones((LANES,), jnp.float32)

        def fill(val):
            def row(r, carry):
                for q in range(h_dim // LANES):
                    ones_v[r, pl.ds(q * LANES, LANES)] = val
                return carry
            lax.fori_loop(0, GSZ, row, 0)

        fill(zero16)

        def zchunk(t, carry):
            pltpu.sync_copy(ones_v, cnt_acc.at[pl.ds(base + t * GSZ, GSZ)])
            return carry
        lax.fori_loop(0, n_full, zchunk, 0)
        if rem:
            pltpu.sync_copy(ones_v.at[pl.ds(0, rem)],
                            cnt_acc.at[pl.ds(base + n_full * GSZ, rem)])

        fill(one16)

        pltpu.sync_copy(dst_hbm.at[wid], dst_v)

        plsc.subcore_barrier()

        for phase in range(SPLIT):
            def grp(j, carry):
                pltpu.sync_copy(ones_v, cnt_acc.at[dst_v.at[phase, j]],
                                add=True)
                return carry
            lax.fori_loop(0, k4, grp, 0)

        plsc.subcore_barrier()

        def ochunk(t, carry):
            pltpu.sync_copy(cnt_acc.at[pl.ds(base + t * GSZ, GSZ)],
                            cnts_hbm.at[core, pl.ds(base + t * GSZ, GSZ)])
            return carry
        lax.fori_loop(0, n_full, ochunk, 0)
        if rem:
            pltpu.sync_copy(cnt_acc.at[pl.ds(base + n_full * GSZ, rem)],
                            cnts_hbm.at[core, pl.ds(base + n_full * GSZ, rem)])

    fn = pl.kernel(
        body,
        out_type=jax.ShapeDtypeStruct((NC, n_pad, h_dim), jnp.float32),
        mesh=_mesh(),
        scratch_types=[
            pltpu.VMEM((SPLIT, k4, GSZ), jnp.int32),
            pltpu.VMEM((GSZ, h_dim), jnp.float32),
            pltpu.VMEM_SHARED((n_pad, h_dim), jnp.float32),
            pltpu.SemaphoreType.DMA,
        ])
    return fn(dst4)


def _sc_sums(h, src4, dst4, n_pad):
    k4 = src4.shape[2]
    h_dim = h.shape[1]
    rpt = n_pad // NS
    n_full = rpt // SGSZ
    rem = rpt - n_full * SGSZ

    def body(h_hbm, src_hbm, dst_hbm, sums_hbm, src_v, dst_v,
             msg0, msg1, acc, gs0, gs1, ss0, ss1):
        core = lax.axis_index("c")
        sub = lax.axis_index("s")
        wid = sub * NC + core
        base = sub * rpt
        msgs = (msg0, msg1)
        gsems = (gs0, gs1)
        ssems = (ss0, ss1)

        zero16 = jnp.zeros((LANES,), jnp.float32)

        def zrow(r, carry):
            for q in range(h_dim // LANES):
                msg0[r, pl.ds(q * LANES, LANES)] = zero16
            return carry
        lax.fori_loop(0, SGSZ, zrow, 0)

        def zchunk(t, carry):
            pltpu.sync_copy(msg0, acc.at[pl.ds(base + t * SGSZ, SGSZ)])
            return carry
        lax.fori_loop(0, n_full, zchunk, 0)
        if rem:
            pltpu.sync_copy(msg0.at[pl.ds(0, rem)],
                            acc.at[pl.ds(base + n_full * SGSZ, rem)])

        plsc.subcore_barrier()

        for phase in range(SPLIT):
            pltpu.sync_copy(src_hbm.at[wid, phase], src_v)
            pltpu.sync_copy(dst_hbm.at[wid, phase], dst_v)

            for b in range(NBUF):
                pltpu.async_copy(h_hbm.at[src_v.at[b]], msgs[b], gsems[b])

            def trip(p, carry):
                j = NBUF * p
                for b in range(NBUF):
                    pltpu.make_async_copy(
                        h_hbm.at[src_v.at[j + b]], msgs[b], gsems[b]).wait()
                    pltpu.async_copy(
                        msgs[b], acc.at[dst_v.at[j + b]], ssems[b], add=True)
                for b in range(NBUF):
                    jn = jnp.minimum(j + b + NBUF, k4 - 1)
                    pltpu.make_async_copy(
                        msgs[b], acc.at[dst_v.at[j + b]], ssems[b]).wait()
                    pltpu.async_copy(h_hbm.at[src_v.at[jn]], msgs[b], gsems[b])
                return carry
            lax.fori_loop(0, k4 // NBUF, trip, 0)

            for b in range(NBUF):
                pltpu.make_async_copy(
                    h_hbm.at[src_v.at[k4 - 1]], msgs[b], gsems[b]).wait()

        plsc.subcore_barrier()

        def ochunk(t, carry):
            pltpu.sync_copy(acc.at[pl.ds(base + t * SGSZ, SGSZ)],
                            sums_hbm.at[core, pl.ds(base + t * SGSZ, SGSZ)])
            return carry
        lax.fori_loop(0, n_full, ochunk, 0)
        if rem:
            pltpu.sync_copy(acc.at[pl.ds(base + n_full * SGSZ, rem)],
                            sums_hbm.at[core, pl.ds(base + n_full * SGSZ, rem)])

    fn = pl.kernel(
        body,
        out_type=jax.ShapeDtypeStruct((NC, n_pad, h_dim), jnp.float32),
        mesh=_mesh(),
        scratch_types=[
            pltpu.VMEM((k4, SGSZ), jnp.int32),
            pltpu.VMEM((k4, SGSZ), jnp.int32),
            pltpu.VMEM((SGSZ, h_dim), jnp.float32),
            pltpu.VMEM((SGSZ, h_dim), jnp.float32),
            pltpu.VMEM_SHARED((n_pad, h_dim), jnp.float32),
            pltpu.SemaphoreType.DMA,
            pltpu.SemaphoreType.DMA,
            pltpu.SemaphoreType.DMA,
            pltpu.SemaphoreType.DMA,
        ])
    return fn(h, src4, dst4)


def kernel(x, edge_index, W_feat, b_feat, group_encodings, W_agg1, b_agg1,
           W_agg2, b_agg2, W_c1, b_c1, W_c2, b_c2):
    n, _ = x.shape
    e = edge_index.shape[1]
    n_pad = ((n + 1) + NS * 8 - 1) // (NS * 8) * (NS * 8)

    src = edge_index[0]
    dst = edge_index[1]

    kc = -(-e // (NW * GSZ))
    kc = (kc + SPLIT - 1) // SPLIT * SPLIT
    ar_c = jnp.arange(NW * GSZ * kc - e, dtype=jnp.int32)
    dst4c = jnp.concatenate([dst, n + ar_c % (n_pad - n)]).reshape(
        NW, SPLIT, kc // SPLIT, GSZ)

    q = SPLIT * NBUF
    ks = -(-e // (NW * SGSZ))
    ks = (ks + q - 1) // q * q
    ar_s = jnp.arange(NW * SGSZ * ks - e, dtype=jnp.int32)
    src4s = jnp.concatenate([src, ar_s % n]).reshape(
        NW, SPLIT, ks // SPLIT, SGSZ)
    dst4s = jnp.concatenate([dst, n + ar_s % (n_pad - n)]).reshape(
        NW, SPLIT, ks // SPLIT, SGSZ)

    cnts = _sc_counts(dst4c, n_pad, W_feat.shape[1])
    h = _tc_feat(x, W_feat, b_feat)
    sums1 = _sc_sums(h, src4s, dst4s, n_pad)
    h1 = _tc_mid(sums1, cnts, W_agg1, b_agg1)
    sums2 = _sc_sums(h1, src4s, dst4s, n_pad)
    out = _tc_final(sums2, cnts, W_agg2, b_agg2, group_encodings,
                    W_c1, b_c1, W_c2, b_c2)
    return out[:n]


# speedup vs baseline: 1.2568x; 1.2568x over previous
import jax
import jax.numpy as jnp
from jax import lax
from jax.experimental import pallas as pl
from jax.experimental.pallas import tpu as pltpu
from jax.experimental.pallas import tpu_sc as plsc

NC = 2
NS = 16
NW = NC * NS
LANES = 16
GSZ = 128
SGSZ = 128
NBUF = 2
SPLIT = 2


def _feat_body(x_ref, w_ref, b_ref, o_ref):
    acc = jnp.dot(x_ref[...], w_ref[...], preferred_element_type=jnp.float32)
    o_ref[...] = jnp.maximum(acc + b_ref[...], 0.0)


def _tc_feat(x, w, b):
    n, d = x.shape
    h_dim = w.shape[1]
    bn = 1000
    return pl.pallas_call(
        _feat_body,
        grid=(n // bn,),
        in_specs=[
            pl.BlockSpec((bn, d), lambda i: (i, 0)),
            pl.BlockSpec((d, h_dim), lambda i: (0, 0)),
            pl.BlockSpec((1, h_dim), lambda i: (0, 0)),
        ],
        out_specs=pl.BlockSpec((bn, h_dim), lambda i: (i, 0)),
        out_shape=jax.ShapeDtypeStruct((n, h_dim), jnp.float32),
    )(x, w, b.reshape(1, -1))


def _mid_body(s_ref, c_ref, w_ref, b_ref, o_ref):
    cnt = c_ref[0, :, :1] + c_ref[1, :, :1]
    m = (s_ref[0] + s_ref[1]) / jnp.maximum(cnt, 1.0)
    acc = jnp.dot(m, w_ref[...], preferred_element_type=jnp.float32)
    o_ref[...] = jnp.maximum(acc + b_ref[...], 0.0)


def _tc_mid(sums, cnts, w, b):
    _, n_pad, h_dim = sums.shape
    bn = n_pad // 4
    return pl.pallas_call(
        _mid_body,
        grid=(4,),
        in_specs=[
            pl.BlockSpec((2, bn, h_dim), lambda i: (0, i, 0)),
            pl.BlockSpec((2, bn, h_dim), lambda i: (0, i, 0)),
            pl.BlockSpec((h_dim, h_dim), lambda i: (0, 0)),
            pl.BlockSpec((1, h_dim), lambda i: (0, 0)),
        ],
        out_specs=pl.BlockSpec((bn, h_dim), lambda i: (i, 0)),
        out_shape=jax.ShapeDtypeStruct((n_pad, h_dim), jnp.float32),
    )(sums, cnts, w, b.reshape(1, -1))


def _final_body(s_ref, c_ref, w2_ref, b2_ref, ge_ref, wc1_ref, bc1_ref,
                wc2_ref, bc2_ref, o_ref):
    cnt = c_ref[0, :, :1] + c_ref[1, :, :1]
    m = (s_ref[0] + s_ref[1]) / jnp.maximum(cnt, 1.0)
    h2 = jnp.dot(m, w2_ref[...], preferred_element_type=jnp.float32) + b2_ref[...]
    h2 = h2 + jnp.mean(ge_ref[...], axis=0, keepdims=True)
    hid = jnp.dot(h2, wc1_ref[...], preferred_element_type=jnp.float32) + bc1_ref[...]
    hid = jnp.maximum(hid, 0.0)
    o_ref[...] = jnp.dot(hid, wc2_ref[...], preferred_element_type=jnp.float32) + bc2_ref[...]


def _tc_final(sums, cnts, w2, b2, ge, wc1, bc1, wc2, bc2):
    _, n_pad, h_dim = sums.shape
    g = ge.shape[0]
    hid_dim = wc1.shape[1]
    out_dim = wc2.shape[1]
    bn = n_pad // 4
    return pl.pallas_call(
        _final_body,
        grid=(4,),
        in_specs=[
            pl.BlockSpec((2, bn, h_dim), lambda i: (0, i, 0)),
            pl.BlockSpec((2, bn, h_dim), lambda i: (0, i, 0)),
            pl.BlockSpec((h_dim, h_dim), lambda i: (0, 0)),
            pl.BlockSpec((1, h_dim), lambda i: (0, 0)),
            pl.BlockSpec((g, h_dim), lambda i: (0, 0)),
            pl.BlockSpec((h_dim, hid_dim), lambda i: (0, 0)),
            pl.BlockSpec((1, hid_dim), lambda i: (0, 0)),
            pl.BlockSpec((hid_dim, out_dim), lambda i: (0, 0)),
            pl.BlockSpec((1, out_dim), lambda i: (0, 0)),
        ],
        out_specs=pl.BlockSpec((bn, out_dim), lambda i: (i, 0)),
        out_shape=jax.ShapeDtypeStruct((n_pad, out_dim), jnp.float32),
    )(sums, cnts, w2, b2.reshape(1, -1), ge, wc1, bc1.reshape(1, -1),
      wc2, bc2.reshape(1, -1))


def _mesh():
    return plsc.VectorSubcoreMesh(
        core_axis_name="c", subcore_axis_name="s",
        num_cores=NC, num_subcores=NS)


def _sc_counts(dst4, n_pad, h_dim):
    k4 = dst4.shape[2]
    rpt = n_pad // NS
    n_full = rpt // GSZ
    rem = rpt - n_full * GSZ

    def body(dst_hbm, cnts_hbm, dst_v, ones_v, cnt_acc, sem):
        del sem
        core = lax.axis_index("c")
        sub = lax.axis_index("s")
        wid = sub * NC + core
        base = sub * rpt

        zero16 = jnp.zeros((LANES,), jnp.float32)
        one16 = jnp.ones((LANES,), jnp.float32)

        def fill(val):
            def row(r, carry):
                for q in range(h_dim // LANES):
                    ones_v[r, pl.ds(q * LANES, LANES)] = val
                return carry
            lax.fori_loop(0, GSZ, row, 0)

        fill(zero16)

        def zchunk(t, carry):
            pltpu.sync_copy(ones_v, cnt_acc.at[pl.ds(base + t * GSZ, GSZ)])
            return carry
        lax.fori_loop(0, n_full, zchunk, 0)
        if rem:
            pltpu.sync_copy(ones_v.at[pl.ds(0, rem)],
                            cnt_acc.at[pl.ds(base + n_full * GSZ, rem)])

        fill(one16)

        pltpu.sync_copy(dst_hbm.at[wid], dst_v)

        plsc.subcore_barrier()

        for phase in range(SPLIT):
            def grp(j, carry):
                pltpu.sync_copy(ones_v, cnt_acc.at[dst_v.at[phase, j]],
                                add=True)
                return carry
            lax.fori_loop(0, k4, grp, 0)

        plsc.subcore_barrier()

        def ochunk(t, carry):
            pltpu.sync_copy(cnt_acc.at[pl.ds(base + t * GSZ, GSZ)],
                            cnts_hbm.at[core, pl.ds(base + t * GSZ, GSZ)])
            return carry
        lax.fori_loop(0, n_full, ochunk, 0)
        if rem:
            pltpu.sync_copy(cnt_acc.at[pl.ds(base + n_full * GSZ, rem)],
                            cnts_hbm.at[core, pl.ds(base + n_full * GSZ, rem)])

    fn = pl.kernel(
        body,
        out_type=jax.ShapeDtypeStruct((NC, n_pad, h_dim), jnp.float32),
        mesh=_mesh(),
        scratch_types=[
            pltpu.VMEM((SPLIT, k4, GSZ), jnp.int32),
            pltpu.VMEM((GSZ, h_dim), jnp.float32),
            pltpu.VMEM_SHARED((n_pad, h_dim), jnp.float32),
            pltpu.SemaphoreType.DMA,
        ])
    return fn(dst4)


def _sc_sums(h, src4, dst4, n_pad):
    k4 = src4.shape[2]
    h_dim = h.shape[1]
    rpt = n_pad // NS
    n_full = rpt // SGSZ
    rem = rpt - n_full * SGSZ

    def body(h_hbm, src_hbm, dst_hbm, sums_hbm, src_v, dst_v,
             msg0, msg1, acc, gs0, gs1):
        core = lax.axis_index("c")
        sub = lax.axis_index("s")
        wid = sub * NC + core
        base = sub * rpt
        msgs = (msg0, msg1)
        gsems = (gs0, gs1)

        zero16 = jnp.zeros((LANES,), jnp.float32)

        def zrow(r, carry):
            for q in range(h_dim // LANES):
                msg0[r, pl.ds(q * LANES, LANES)] = zero16
            return carry
        lax.fori_loop(0, SGSZ, zrow, 0)

        def zchunk(t, carry):
            pltpu.sync_copy(msg0, acc.at[pl.ds(base + t * SGSZ, SGSZ)])
            return carry
        lax.fori_loop(0, n_full, zchunk, 0)
        if rem:
            pltpu.sync_copy(msg0.at[pl.ds(0, rem)],
                            acc.at[pl.ds(base + n_full * SGSZ, rem)])

        plsc.subcore_barrier()

        for phase in range(SPLIT):
            pltpu.sync_copy(src_hbm.at[wid, phase], src_v)
            pltpu.sync_copy(dst_hbm.at[wid, phase], dst_v)

            for b in range(NBUF):
                pltpu.async_copy(h_hbm.at[src_v.at[b]], msgs[b], gsems[b])

            def trip(p, carry):
                j = NBUF * p
                for b in range(NBUF):
                    jn = jnp.minimum(j + b + NBUF, k4 - 1)
                    pltpu.make_async_copy(
                        h_hbm.at[src_v.at[j + b]], msgs[b], gsems[b]).wait()
                    pltpu.sync_copy(msgs[b], acc.at[dst_v.at[j + b]], add=True)
                    pltpu.async_copy(h_hbm.at[src_v.at[jn]], msgs[b], gsems[b])
                return carry
            lax.fori_loop(0, k4 // NBUF, trip, 0)

            for b in range(NBUF):
                pltpu.make_async_copy(
                    h_hbm.at[src_v.at[k4 - 1]], msgs[b], gsems[b]).wait()

        plsc.subcore_barrier()

        def ochunk(t, carry):
            pltpu.sync_copy(acc.at[pl.ds(base + t * SGSZ, SGSZ)],
                            sums_hbm.at[core, pl.ds(base + t * SGSZ, SGSZ)])
            return carry
        lax.fori_loop(0, n_full, ochunk, 0)
        if rem:
            pltpu.sync_copy(acc.at[pl.ds(base + n_full * SGSZ, rem)],
                            sums_hbm.at[core, pl.ds(base + n_full * SGSZ, rem)])

    fn = pl.kernel(
        body,
        out_type=jax.ShapeDtypeStruct((NC, n_pad, h_dim), jnp.float32),
        mesh=_mesh(),
        scratch_types=[
            pltpu.VMEM((k4, SGSZ), jnp.int32),
            pltpu.VMEM((k4, SGSZ), jnp.int32),
            pltpu.VMEM((SGSZ, h_dim), jnp.float32),
            pltpu.VMEM((SGSZ, h_dim), jnp.float32),
            pltpu.VMEM_SHARED((n_pad, h_dim), jnp.float32),
            pltpu.SemaphoreType.DMA,
            pltpu.SemaphoreType.DMA,
        ])
    return fn(h, src4, dst4)


def kernel(x, edge_index, W_feat, b_feat, group_encodings, W_agg1, b_agg1,
           W_agg2, b_agg2, W_c1, b_c1, W_c2, b_c2):
    n, _ = x.shape
    e = edge_index.shape[1]
    n_pad = ((n + 1) + NS * 8 - 1) // (NS * 8) * (NS * 8)

    src = edge_index[0]
    dst = edge_index[1]

    kc = -(-e // (NW * GSZ))
    kc = (kc + SPLIT - 1) // SPLIT * SPLIT
    ar_c = jnp.arange(NW * GSZ * kc - e, dtype=jnp.int32)
    dst4c = jnp.concatenate([dst, n + ar_c % (n_pad - n)]).reshape(
        NW, SPLIT, kc // SPLIT, GSZ)

    q = SPLIT * NBUF
    ks = -(-e // (NW * SGSZ))
    ks = (ks + q - 1) // q * q
    ar_s = jnp.arange(NW * SGSZ * ks - e, dtype=jnp.int32)
    src4s = jnp.concatenate([src, ar_s % n]).reshape(
        NW, SPLIT, ks // SPLIT, SGSZ)
    dst4s = jnp.concatenate([dst, n + ar_s % (n_pad - n)]).reshape(
        NW, SPLIT, ks // SPLIT, SGSZ)

    cnts = _sc_counts(dst4c, n_pad, W_feat.shape[1])
    h = _tc_feat(x, W_feat, b_feat)
    sums1 = _sc_sums(h, src4s, dst4s, n_pad)
    h1 = _tc_mid(sums1, cnts, W_agg1, b_agg1)
    sums2 = _sc_sums(h1, src4s, dst4s, n_pad)
    out = _tc_final(sums2, cnts, W_agg2, b_agg2, group_encodings,
                    W_c1, b_c1, W_c2, b_c2)
    return out[:n]
